# Initial kernel scaffold; baseline (speedup 1.0000x reference)
import jax
import jax.numpy as jnp
from jax.experimental import pallas as pl


def kernel(users, pos_items, edge_index, A_values, user_embed, entity_embed, W_gc_0, b_gc_0, W_bi_0, b_bi_0, W_gc_1, b_gc_1, W_bi_1, b_bi_1):
    raise NotImplementedError("write your pallas kernel here")


# speedup vs baseline: 2.5186x; 2.5186x over previous
import dataclasses
import functools

import jax
import jax.numpy as jnp
from jax import lax
from jax.experimental import pallas as pl
from jax.experimental.pallas import tpu as pltpu
from jax.experimental.pallas import tpu_sc as plsc

N_USERS_C = 2000
N_C = 10000
D_C = 128
E_C = 320000
B_C = 1024

NC = 2
NS = 16
NW = NC * NS
CH = 128
EPT = 10240
E_PAD = EPT * NW
NCHUNK = EPT // CH
NP_C = 10240
RPS = NP_C // NS
BPT = B_C // NW

_mesh = plsc.VectorSubcoreMesh(core_axis_name="c", subcore_axis_name="s")
_f32 = jnp.float32
_i32 = jnp.int32

_sc_params = pltpu.CompilerParams()
if "needs_layout_passes" in pltpu.CompilerParams.__dataclass_fields__:
  _sc_params = dataclasses.replace(_sc_params, needs_layout_passes=False)


def _side_sc(ego, src_t, dst_t, a_t, zeros):
  @functools.partial(
      pl.kernel,
      out_type=jax.ShapeDtypeStruct((NC, NP_C, D_C), _f32),
      mesh=_mesh,
      scratch_types=[
          pltpu.VMEM((NCHUNK, CH), _i32),
          pltpu.VMEM((NCHUNK, CH), _i32),
          pltpu.VMEM((EPT,), _f32),
          pltpu.VMEM((CH, D_C), _f32),
          pltpu.VMEM_SHARED((NP_C, D_C), _f32),
          pltpu.SemaphoreType.DMA,
      ],
      compiler_params=_sc_params,
  )
  def k(ego_h, src_h, dst_h, a_h, z_h, out_h, src_v, dst_v, a_v, rows_v,
        acc_sh, sem):
    cid = lax.axis_index("c")
    sid = lax.axis_index("s")
    wid = sid * NC + cid
    pltpu.sync_copy(z_h.at[pl.ds(sid * RPS, RPS)],
                    acc_sh.at[pl.ds(sid * RPS, RPS)])
    pltpu.sync_copy(src_h.at[wid], src_v)
    pltpu.sync_copy(dst_h.at[wid], dst_v)
    pltpu.sync_copy(a_h.at[wid], a_v)
    plsc.subcore_barrier()

    @pl.loop(0, NCHUNK)
    def _chunk(j):
      pltpu.async_copy(ego_h.at[src_v.at[j]], rows_v, sem).wait()

      @pl.loop(0, CH)
      def _edge(e):
        a_vec = plsc.load_gather(a_v, [jnp.full((16,), j * CH + e, _i32)])
        for r in range(D_C // 16):
          sl = rows_v.at[e, pl.ds(r * 16, 16)]
          sl[...] = sl[...] * a_vec

      pltpu.sync_copy(rows_v, acc_sh.at[dst_v.at[j]], add=True)

    plsc.subcore_barrier()
    pltpu.sync_copy(acc_sh.at[pl.ds(sid * RPS, RPS)],
                    out_h.at[cid, pl.ds(sid * RPS, RPS)])

  return k(ego, src_t, dst_t, a_t, zeros)


def _dense_tc(ego, parts, Wg, bg, Wb, bb):
  blk = 1000
  grid = N_C // blk

  def body(ego_r, p_r, wg_r, bg_r, wb_r, bb_r, new_r, nrm_r):
    side = p_r[0] + p_r[1]
    e = ego_r[...]
    s = jnp.dot(e + side, wg_r[...], preferred_element_type=_f32,
                precision=lax.Precision.HIGHEST) + bg_r[...]
    s = jnp.where(s >= 0, s, 0.2 * s)
    m = jnp.dot(e * side, wb_r[...], preferred_element_type=_f32,
                precision=lax.Precision.HIGHEST) + bb_r[...]
    m = jnp.where(m >= 0, m, 0.2 * m)
    out = s + m
    new_r[...] = out
    norm = jnp.sqrt(jnp.sum(out * out, axis=1, keepdims=True))
    nrm_r[...] = out / jnp.maximum(norm, 1e-12)

  return pl.pallas_call(
      body,
      grid=(grid,),
      in_specs=[
          pl.BlockSpec((blk, D_C), lambda i: (i, 0)),
          pl.BlockSpec((NC, blk, D_C), lambda i: (0, i, 0)),
          pl.BlockSpec((D_C, D_C), lambda i: (0, 0)),
          pl.BlockSpec((1, D_C), lambda i: (0, 0)),
          pl.BlockSpec((D_C, D_C), lambda i: (0, 0)),
          pl.BlockSpec((1, D_C), lambda i: (0, 0)),
      ],
      out_specs=[
          pl.BlockSpec((blk, D_C), lambda i: (i, 0)),
          pl.BlockSpec((blk, D_C), lambda i: (i, 0)),
      ],
      out_shape=[
          jax.ShapeDtypeStruct((N_C, D_C), _f32),
          jax.ShapeDtypeStruct((N_C, D_C), _f32),
      ],
  )(ego, parts, Wg, bg, Wb, bb)


def _gather_sc(t0, t1, t2, u_t, p_t):
  @functools.partial(
      pl.kernel,
      out_type=(jax.ShapeDtypeStruct((3, B_C, D_C), _f32),
                jax.ShapeDtypeStruct((3, B_C, D_C), _f32)),
      mesh=_mesh,
      scratch_types=[
          pltpu.VMEM((BPT,), _i32),
          pltpu.VMEM((BPT, D_C), _f32),
          pltpu.SemaphoreType.DMA,
      ],
  )
  def k(t0_h, t1_h, t2_h, u_h, p_h, ue_h, pe_h, idx_v, rows_v, sem):
    cid = lax.axis_index("c")
    sid = lax.axis_index("s")
    wid = sid * NC + cid
    for idx_h, out_h in ((u_h, ue_h), (p_h, pe_h)):
      pltpu.sync_copy(idx_h.at[wid], idx_v)
      for t, t_h in enumerate((t0_h, t1_h, t2_h)):
        pltpu.async_copy(t_h.at[idx_v], rows_v, sem).wait()
        pltpu.sync_copy(rows_v, out_h.at[t, pl.ds(wid * BPT, BPT)])

  return k(t0, t1, t2, u_t, p_t)


def _score_tc(u_parts, p_parts):
  def body(u_r, p_r, o_r):
    acc = jnp.zeros((B_C, B_C), _f32)
    for t in range(3):
      acc = acc + lax.dot_general(
          u_r[t], p_r[t], (((1,), (1,)), ((), ())),
          preferred_element_type=_f32, precision=lax.Precision.HIGHEST)
    o_r[...] = acc

  return pl.pallas_call(
      body,
      out_shape=jax.ShapeDtypeStruct((B_C, B_C), _f32),
  )(u_parts, p_parts)


def kernel(users, pos_items, edge_index, A_values, user_embed, entity_embed,
           W_gc_0, b_gc_0, W_bi_0, b_bi_0, W_gc_1, b_gc_1, W_bi_1, b_bi_1):
  ego0 = jnp.concatenate([user_embed, entity_embed], axis=0)
  zeros = jnp.zeros((NP_C, D_C), _f32)

  pad = E_PAD - E_C
  src_t = jnp.pad(edge_index[0].astype(_i32), (0, pad)).reshape(NW, NCHUNK, CH)
  dst_t = jnp.pad(edge_index[1].astype(_i32), (0, pad)).reshape(NW, NCHUNK, CH)
  a_t = jnp.pad(A_values.astype(_f32), (0, pad)).reshape(NW, EPT)

  parts1 = _side_sc(ego0, src_t, dst_t, a_t, zeros)
  ego1, n1 = _dense_tc(ego0, parts1, W_gc_0, b_gc_0, W_bi_0, b_bi_0)
  parts2 = _side_sc(ego1, src_t, dst_t, a_t, zeros)
  _, n2 = _dense_tc(ego1, parts2, W_gc_1, b_gc_1, W_bi_1, b_bi_1)

  u_t = users.astype(_i32).reshape(NW, BPT)
  p_t = (pos_items.astype(_i32) + N_USERS_C).reshape(NW, BPT)
  u_parts, p_parts = _gather_sc(ego0, n1, n2, u_t, p_t)
  return _score_tc(u_parts, p_parts)


# speedup vs baseline: 3.3899x; 1.3459x over previous
import dataclasses
import functools

import jax
import jax.numpy as jnp
from jax import lax
from jax.experimental import pallas as pl
from jax.experimental.pallas import tpu as pltpu
from jax.experimental.pallas import tpu_sc as plsc

N_USERS_C = 2000
N_C = 10000
D_C = 128
E_C = 320000
B_C = 1024

NC = 2
NS = 16
NW = NC * NS
CH = 128
NBUF = 2
NIDX = 4
EPT = 10240
E_PAD = EPT * NW
NCHUNK = EPT // CH
NP_C = 10240
RPS = NP_C // NS
BPT = B_C // NW

_mesh = plsc.VectorSubcoreMesh(core_axis_name="c", subcore_axis_name="s")
_f32 = jnp.float32
_i32 = jnp.int32

_sc_params = pltpu.CompilerParams()
if "needs_layout_passes" in pltpu.CompilerParams.__dataclass_fields__:
  _sc_params = dataclasses.replace(_sc_params, needs_layout_passes=False)


def _side_sc(ego, ed, zeros):
  @functools.partial(
      pl.kernel,
      out_type=jax.ShapeDtypeStruct((NC, NP_C, D_C), _f32),
      mesh=_mesh,
      scratch_types=[
          [pltpu.VMEM((3, CH), _i32) for _ in range(NIDX)],
          [pltpu.VMEM((CH, D_C), _f32) for _ in range(NBUF)],
          pltpu.VMEM_SHARED((NP_C, D_C), _f32),
          [pltpu.SemaphoreType.DMA for _ in range(NBUF)],
          [pltpu.SemaphoreType.DMA for _ in range(NBUF)],
          [pltpu.SemaphoreType.DMA for _ in range(NIDX)],
      ],
      compiler_params=_sc_params,
  )
  def k(ego_h, ed_h, z_h, out_h, ed_v, rows, acc_sh, gsem, ssem, isem):
    cid = lax.axis_index("c")
    sid = lax.axis_index("s")
    wid = sid * NC + cid
    pltpu.sync_copy(z_h.at[pl.ds(sid * RPS, RPS)],
                    acc_sh.at[pl.ds(sid * RPS, RPS)])

    def istart(s, c):
      pltpu.async_copy(ed_h.at[wid, c], ed_v[s], isem[s])

    def iwait(s, c):
      pltpu.make_async_copy(ed_h.at[wid, c], ed_v[s], isem[s]).wait()

    def gstart(b, s):
      pltpu.async_copy(ego_h.at[ed_v[s].at[0]], rows[b], gsem[b])

    def gwait(b, s):
      pltpu.make_async_copy(ego_h.at[ed_v[s].at[0]], rows[b], gsem[b]).wait()

    def sstart(b, s):
      pltpu.async_copy(rows[b], acc_sh.at[ed_v[s].at[1]], ssem[b], add=True)

    def swait(b, s):
      pltpu.make_async_copy(rows[b], acc_sh.at[ed_v[s].at[1]],
                            ssem[b]).wait()

    def scale(b, s):
      row2 = jnp.full((16,), 2, _i32)

      @pl.loop(0, CH)
      def _edge(e):
        bits = plsc.load_gather(ed_v[s], [row2, jnp.full((16,), e, _i32)])
        a_vec = plsc.bitcast(bits, _f32)
        for r in range(D_C // 16):
          sl = rows[b].at[e, pl.ds(r * 16, 16)]
          sl[...] = sl[...] * a_vec

    for s in range(NIDX):
      istart(s, s)
    for b in range(NBUF):
      iwait(b, b)
      gstart(b, b)

    @pl.loop(0, NCHUNK, step=NIDX)
    def _chunks(j):
      for b in range(NIDX):
        c = j + b
        bb = b % NBUF
        gwait(bb, b)
        scale(bb, b)
        sstart(bb, b)

        @pl.when(c + NBUF < NCHUNK)
        def _refill():
          swait(bb, b)
          iwait((b + NBUF) % NIDX, c + NBUF)
          gstart(bb, (b + NBUF) % NIDX)

          @pl.when(c + NIDX < NCHUNK)
          def _meta():
            istart(b, c + NIDX)

    for x in (NCHUNK - 2, NCHUNK - 1):
      swait(x % NBUF, x % NIDX)
    plsc.subcore_barrier()
    pltpu.sync_copy(acc_sh.at[pl.ds(sid * RPS, RPS)],
                    out_h.at[cid, pl.ds(sid * RPS, RPS)])

  return k(ego, ed, zeros)


def _dense_tc(ego, parts, Wg, bg, Wb, bb):
  blk = 1000
  grid = N_C // blk

  def body(ego_r, p_r, wg_r, bg_r, wb_r, bb_r, new_r, nrm_r):
    side = p_r[0] + p_r[1]
    e = ego_r[...]
    s = jnp.dot(e + side, wg_r[...], preferred_element_type=_f32,
                precision=lax.Precision.HIGHEST) + bg_r[...]
    s = jnp.where(s >= 0, s, 0.2 * s)
    m = jnp.dot(e * side, wb_r[...], preferred_element_type=_f32,
                precision=lax.Precision.HIGHEST) + bb_r[...]
    m = jnp.where(m >= 0, m, 0.2 * m)
    out = s + m
    new_r[...] = out
    norm = jnp.sqrt(jnp.sum(out * out, axis=1, keepdims=True))
    nrm_r[...] = out / jnp.maximum(norm, 1e-12)

  return pl.pallas_call(
      body,
      grid=(grid,),
      in_specs=[
          pl.BlockSpec((blk, D_C), lambda i: (i, 0)),
          pl.BlockSpec((NC, blk, D_C), lambda i: (0, i, 0)),
          pl.BlockSpec((D_C, D_C), lambda i: (0, 0)),
          pl.BlockSpec((1, D_C), lambda i: (0, 0)),
          pl.BlockSpec((D_C, D_C), lambda i: (0, 0)),
          pl.BlockSpec((1, D_C), lambda i: (0, 0)),
      ],
      out_specs=[
          pl.BlockSpec((blk, D_C), lambda i: (i, 0)),
          pl.BlockSpec((blk, D_C), lambda i: (i, 0)),
      ],
      out_shape=[
          jax.ShapeDtypeStruct((N_C, D_C), _f32),
          jax.ShapeDtypeStruct((N_C, D_C), _f32),
      ],
  )(ego, parts, Wg, bg, Wb, bb)


def _gather_sc(t0, t1, t2, u_t, p_t):
  @functools.partial(
      pl.kernel,
      out_type=(jax.ShapeDtypeStruct((3, B_C, D_C), _f32),
                jax.ShapeDtypeStruct((3, B_C, D_C), _f32)),
      mesh=_mesh,
      scratch_types=[
          pltpu.VMEM((BPT,), _i32),
          pltpu.VMEM((BPT, D_C), _f32),
          pltpu.SemaphoreType.DMA,
      ],
  )
  def k(t0_h, t1_h, t2_h, u_h, p_h, ue_h, pe_h, idx_v, rows_v, sem):
    cid = lax.axis_index("c")
    sid = lax.axis_index("s")
    wid = sid * NC + cid
    for idx_h, out_h in ((u_h, ue_h), (p_h, pe_h)):
      pltpu.sync_copy(idx_h.at[wid], idx_v)
      for t, t_h in enumerate((t0_h, t1_h, t2_h)):
        pltpu.async_copy(t_h.at[idx_v], rows_v, sem).wait()
        pltpu.sync_copy(rows_v, out_h.at[t, pl.ds(wid * BPT, BPT)])

  return k(t0, t1, t2, u_t, p_t)


def _score_tc(u_parts, p_parts):
  def body(u_r, p_r, o_r):
    acc = jnp.zeros((B_C, B_C), _f32)
    for t in range(3):
      acc = acc + lax.dot_general(
          u_r[t], p_r[t], (((1,), (1,)), ((), ())),
          preferred_element_type=_f32, precision=lax.Precision.HIGHEST)
    o_r[...] = acc

  return pl.pallas_call(
      body,
      out_shape=jax.ShapeDtypeStruct((B_C, B_C), _f32),
  )(u_parts, p_parts)


def kernel(users, pos_items, edge_index, A_values, user_embed, entity_embed,
           W_gc_0, b_gc_0, W_bi_0, b_bi_0, W_gc_1, b_gc_1, W_bi_1, b_bi_1):
  ego0 = jnp.concatenate([user_embed, entity_embed], axis=0)
  zeros = jnp.zeros((NP_C, D_C), _f32)

  pad = E_PAD - E_C
  src_t = jnp.pad(edge_index[0].astype(_i32), (0, pad)).reshape(NW, NCHUNK, CH)
  dst_t = jnp.pad(edge_index[1].astype(_i32), (0, pad)).reshape(NW, NCHUNK, CH)
  a_bits = jax.lax.bitcast_convert_type(
      jnp.pad(A_values.astype(_f32), (0, pad)), _i32).reshape(NW, NCHUNK, CH)
  ed = jnp.stack([src_t, dst_t, a_bits], axis=2)

  parts1 = _side_sc(ego0, ed, zeros)
  ego1, n1 = _dense_tc(ego0, parts1, W_gc_0, b_gc_0, W_bi_0, b_bi_0)
  parts2 = _side_sc(ego1, ed, zeros)
  _, n2 = _dense_tc(ego1, parts2, W_gc_1, b_gc_1, W_bi_1, b_bi_1)

  u_t = users.astype(_i32).reshape(NW, BPT)
  p_t = (pos_items.astype(_i32) + N_USERS_C).reshape(NW, BPT)
  u_parts, p_parts = _gather_sc(ego0, n1, n2, u_t, p_t)
  return _score_tc(u_parts, p_parts)


# speedup vs baseline: 4.5141x; 1.3317x over previous
import dataclasses
import functools

import jax
import jax.numpy as jnp
from jax import lax
from jax.experimental import pallas as pl
from jax.experimental.pallas import tpu as pltpu
from jax.experimental.pallas import tpu_sc as plsc

N_USERS_C = 2000
N_C = 10000
D_C = 128
E_C = 320000
B_C = 1024

NC = 2
NS = 16
NW = NC * NS
CH = 64
NBUF = 2
NIDX = 8
DW = D_C // 2
EPT = 10240
E_PAD = EPT * NW
NCHUNK = EPT // CH
NP_C = 10240
RPS = NP_C // NS
BPT = B_C // NW

_mesh = plsc.VectorSubcoreMesh(core_axis_name="c", subcore_axis_name="s")
_f32 = jnp.float32
_i32 = jnp.int32

_sc_params = pltpu.CompilerParams()
if "needs_layout_passes" in pltpu.CompilerParams.__dataclass_fields__:
  _sc_params = dataclasses.replace(_sc_params, needs_layout_passes=False)
if "use_tc_tiling_on_sc" in pltpu.CompilerParams.__dataclass_fields__:
  _sc_params = dataclasses.replace(_sc_params, use_tc_tiling_on_sc=False)


def _side_sc(ego_pk, ed, zeros):
  @functools.partial(
      pl.kernel,
      out_type=jax.ShapeDtypeStruct((NC, NP_C, D_C), _f32),
      mesh=_mesh,
      scratch_types=[
          [pltpu.VMEM((3, CH), _i32) for _ in range(NIDX)],
          [pltpu.VMEM((CH, DW), _i32) for _ in range(NBUF)],
          [pltpu.VMEM((CH, D_C), _f32) for _ in range(NBUF)],
          pltpu.VMEM_SHARED((NP_C, D_C), _f32),
          [pltpu.SemaphoreType.DMA for _ in range(NBUF)],
          [pltpu.SemaphoreType.DMA for _ in range(NBUF)],
          [pltpu.SemaphoreType.DMA for _ in range(NIDX)],
      ],
      compiler_params=_sc_params,
  )
  def k(ego_h, ed_h, z_h, out_h, ed_v, pkrows, rows, acc_sh, gsem, ssem, isem):
    cid = lax.axis_index("c")
    sid = lax.axis_index("s")
    wid = sid * NC + cid
    pltpu.sync_copy(z_h.at[pl.ds(sid * RPS, RPS)],
                    acc_sh.at[pl.ds(sid * RPS, RPS)])

    def istart(s, c):
      pltpu.async_copy(ed_h.at[wid, c], ed_v[s], isem[s])

    def iwait(s, c):
      pltpu.make_async_copy(ed_h.at[wid, c], ed_v[s], isem[s]).wait()

    def gstart(b, s):
      pltpu.async_copy(ego_h.at[ed_v[s].at[0]], pkrows[b], gsem[b])

    def gwait(b, s):
      pltpu.make_async_copy(ego_h.at[ed_v[s].at[0]], pkrows[b],
                            gsem[b]).wait()

    def sstart(b, s):
      pltpu.async_copy(rows[b], acc_sh.at[ed_v[s].at[1]], ssem[b], add=True)

    def swait(b, s):
      pltpu.make_async_copy(rows[b], acc_sh.at[ed_v[s].at[1]],
                            ssem[b]).wait()

    def scale(b, s):
      row2 = jnp.full((16,), 2, _i32)

      @pl.loop(0, CH)
      def _edge(e):
        bits = plsc.load_gather(ed_v[s], [row2, jnp.full((16,), e, _i32)])
        a_vec = plsc.bitcast(bits, _f32)
        for g in range(D_C // 32):
          w = pkrows[b][e, pl.ds(g * 16, 16)]
          lo, hi = plsc.unpack(plsc.bitcast(w, jnp.bfloat16),
                               format=plsc.PackFormat.INTERLEAVED)
          rows[b].at[e, pl.ds(g * 32, 16)][...] = lo * a_vec
          rows[b].at[e, pl.ds(g * 32 + 16, 16)][...] = hi * a_vec

    for s in range(NIDX):
      istart(s, s)
    for b in range(NBUF):
      iwait(b, b)
      gstart(b, b)

    @pl.loop(0, NCHUNK, step=NIDX)
    def _chunks(j):
      for b in range(NIDX):
        c = j + b
        bb = b % NBUF
        gwait(bb, b)

        @pl.when(c - 2 >= 0)
        def _drain():
          swait(bb, (b - 2) % NIDX)

        scale(bb, b)
        sstart(bb, b)

        @pl.when(c + NBUF < NCHUNK)
        def _refill():
          iwait((b + NBUF) % NIDX, c + NBUF)
          gstart(bb, (b + NBUF) % NIDX)

        @pl.when(jnp.logical_and(c - 2 >= 0, c + 6 < NCHUNK))
        def _meta():
          istart((b + 6) % NIDX, c + 6)

    for x in (NCHUNK - 2, NCHUNK - 1):
      swait(x % NBUF, x % NIDX)
    plsc.subcore_barrier()
    pltpu.sync_copy(acc_sh.at[pl.ds(sid * RPS, RPS)],
                    out_h.at[cid, pl.ds(sid * RPS, RPS)])

  return k(ego_pk, ed, zeros)


def _dense_tc(ego, parts, Wg, bg, Wb, bb):
  blk = 1000
  grid = N_C // blk

  def body(ego_r, p_r, wg_r, bg_r, wb_r, bb_r, new_r, nrm_r):
    side = p_r[0] + p_r[1]
    e = ego_r[...]
    s = jnp.dot(e + side, wg_r[...], preferred_element_type=_f32,
                precision=lax.Precision.HIGHEST) + bg_r[...]
    s = jnp.where(s >= 0, s, 0.2 * s)
    m = jnp.dot(e * side, wb_r[...], preferred_element_type=_f32,
                precision=lax.Precision.HIGHEST) + bb_r[...]
    m = jnp.where(m >= 0, m, 0.2 * m)
    out = s + m
    new_r[...] = out
    norm = jnp.sqrt(jnp.sum(out * out, axis=1, keepdims=True))
    nrm_r[...] = out / jnp.maximum(norm, 1e-12)

  return pl.pallas_call(
      body,
      grid=(grid,),
      in_specs=[
          pl.BlockSpec((blk, D_C), lambda i: (i, 0)),
          pl.BlockSpec((NC, blk, D_C), lambda i: (0, i, 0)),
          pl.BlockSpec((D_C, D_C), lambda i: (0, 0)),
          pl.BlockSpec((1, D_C), lambda i: (0, 0)),
          pl.BlockSpec((D_C, D_C), lambda i: (0, 0)),
          pl.BlockSpec((1, D_C), lambda i: (0, 0)),
      ],
      out_specs=[
          pl.BlockSpec((blk, D_C), lambda i: (i, 0)),
          pl.BlockSpec((blk, D_C), lambda i: (i, 0)),
      ],
      out_shape=[
          jax.ShapeDtypeStruct((N_C, D_C), _f32),
          jax.ShapeDtypeStruct((N_C, D_C), _f32),
      ],
  )(ego, parts, Wg, bg, Wb, bb)


def _gather_sc(t0, t1, t2, u_t, p_t):
  @functools.partial(
      pl.kernel,
      out_type=(jax.ShapeDtypeStruct((3, B_C, D_C), _f32),
                jax.ShapeDtypeStruct((3, B_C, D_C), _f32)),
      mesh=_mesh,
      scratch_types=[
          pltpu.VMEM((BPT,), _i32),
          pltpu.VMEM((BPT, D_C), _f32),
          pltpu.SemaphoreType.DMA,
      ],
  )
  def k(t0_h, t1_h, t2_h, u_h, p_h, ue_h, pe_h, idx_v, rows_v, sem):
    cid = lax.axis_index("c")
    sid = lax.axis_index("s")
    wid = sid * NC + cid
    for idx_h, out_h in ((u_h, ue_h), (p_h, pe_h)):
      pltpu.sync_copy(idx_h.at[wid], idx_v)
      for t, t_h in enumerate((t0_h, t1_h, t2_h)):
        pltpu.async_copy(t_h.at[idx_v], rows_v, sem).wait()
        pltpu.sync_copy(rows_v, out_h.at[t, pl.ds(wid * BPT, BPT)])

  return k(t0, t1, t2, u_t, p_t)


def _score_tc(u_parts, p_parts):
  def body(u_r, p_r, o_r):
    acc = jnp.zeros((B_C, B_C), _f32)
    for t in range(3):
      acc = acc + lax.dot_general(
          u_r[t], p_r[t], (((1,), (1,)), ((), ())),
          preferred_element_type=_f32, precision=lax.Precision.HIGHEST)
    o_r[...] = acc

  return pl.pallas_call(
      body,
      out_shape=jax.ShapeDtypeStruct((B_C, B_C), _f32),
  )(u_parts, p_parts)


def _pack_table(x):
  bf = x.astype(jnp.bfloat16).reshape(N_C, D_C // 32, 2, 16)
  bf = bf.transpose(0, 1, 3, 2).reshape(N_C, DW, 2)
  return jax.lax.bitcast_convert_type(bf, _i32)


def kernel(users, pos_items, edge_index, A_values, user_embed, entity_embed,
           W_gc_0, b_gc_0, W_bi_0, b_bi_0, W_gc_1, b_gc_1, W_bi_1, b_bi_1):
  ego0 = jnp.concatenate([user_embed, entity_embed], axis=0)
  zeros = jnp.zeros((NP_C, D_C), _f32)

  pad = E_PAD - E_C
  src_t = jnp.pad(edge_index[0].astype(_i32), (0, pad)).reshape(NW, NCHUNK, CH)
  dst_t = jnp.pad(edge_index[1].astype(_i32), (0, pad)).reshape(NW, NCHUNK, CH)
  a_bits = jax.lax.bitcast_convert_type(
      jnp.pad(A_values.astype(_f32), (0, pad)), _i32).reshape(NW, NCHUNK, CH)
  ed = jnp.stack([src_t, dst_t, a_bits], axis=2)

  parts1 = _side_sc(_pack_table(ego0), ed, zeros)
  ego1, n1 = _dense_tc(ego0, parts1, W_gc_0, b_gc_0, W_bi_0, b_bi_0)
  parts2 = _side_sc(_pack_table(ego1), ed, zeros)
  _, n2 = _dense_tc(ego1, parts2, W_gc_1, b_gc_1, W_bi_1, b_bi_1)

  u_t = users.astype(_i32).reshape(NW, BPT)
  p_t = (pos_items.astype(_i32) + N_USERS_C).reshape(NW, BPT)
  u_parts, p_parts = _gather_sc(ego0, n1, n2, u_t, p_t)
  return _score_tc(u_parts, p_parts)


# speedup vs baseline: 4.5513x; 1.0082x over previous
import dataclasses
import functools

import jax
import jax.numpy as jnp
from jax import lax
from jax.experimental import pallas as pl
from jax.experimental.pallas import tpu as pltpu
from jax.experimental.pallas import tpu_sc as plsc

N_USERS_C = 2000
N_C = 10000
D_C = 128
E_C = 320000
B_C = 1024

NC = 2
NS = 16
NW = NC * NS
CH = 64
NBUF = 2
NIDX = 8
DW = D_C // 2
EPT = 10240
E_PAD = EPT * NW
NCHUNK = EPT // CH
NP_C = 10240
RPS = NP_C // NS
BPT = B_C // NW

_mesh = plsc.VectorSubcoreMesh(core_axis_name="c", subcore_axis_name="s")
_f32 = jnp.float32
_i32 = jnp.int32

_sc_params = pltpu.CompilerParams()
if "needs_layout_passes" in pltpu.CompilerParams.__dataclass_fields__:
  _sc_params = dataclasses.replace(_sc_params, needs_layout_passes=False)
if "use_tc_tiling_on_sc" in pltpu.CompilerParams.__dataclass_fields__:
  _sc_params = dataclasses.replace(_sc_params, use_tc_tiling_on_sc=False)


def _side_sc(ego_pk, ed):
  @functools.partial(
      pl.kernel,
      out_type=jax.ShapeDtypeStruct((NC, NP_C, D_C), _f32),
      mesh=_mesh,
      scratch_types=[
          [pltpu.VMEM((3, CH), _i32) for _ in range(NIDX)],
          [pltpu.VMEM((CH, DW), _i32) for _ in range(NBUF)],
          [pltpu.VMEM((CH, D_C), _f32) for _ in range(NBUF)],
          pltpu.VMEM_SHARED((NP_C, D_C), _f32),
          [pltpu.SemaphoreType.DMA for _ in range(NBUF)],
          [pltpu.SemaphoreType.DMA for _ in range(NBUF)],
          [pltpu.SemaphoreType.DMA for _ in range(NIDX)],
      ],
      compiler_params=_sc_params,
  )
  def k(ego_h, ed_h, out_h, ed_v, pkrows, rows, acc_sh, gsem, ssem, isem):
    cid = lax.axis_index("c")
    sid = lax.axis_index("s")
    wid = sid * NC + cid
    zero16 = jnp.zeros((16,), _f32)

    @pl.loop(0, CH)
    def _zrow(r):
      for g in range(D_C // 16):
        rows[0].at[r, pl.ds(g * 16, 16)][...] = zero16

    for kk in range(RPS // CH):
      pltpu.sync_copy(rows[0], acc_sh.at[pl.ds(sid * RPS + kk * CH, CH)])

    def istart(s, c):
      pltpu.async_copy(ed_h.at[wid, c], ed_v[s], isem[s])

    def iwait(s, c):
      pltpu.make_async_copy(ed_h.at[wid, c], ed_v[s], isem[s]).wait()

    def gstart(b, s):
      pltpu.async_copy(ego_h.at[ed_v[s].at[0]], pkrows[b], gsem[b])

    def gwait(b, s):
      pltpu.make_async_copy(ego_h.at[ed_v[s].at[0]], pkrows[b],
                            gsem[b]).wait()

    def sstart(b, s):
      pltpu.async_copy(rows[b], acc_sh.at[ed_v[s].at[1]], ssem[b], add=True)

    def swait(b, s):
      pltpu.make_async_copy(rows[b], acc_sh.at[ed_v[s].at[1]],
                            ssem[b]).wait()

    def scale(b, s):
      row2 = jnp.full((16,), 2, _i32)

      @pl.loop(0, CH)
      def _edge(e):
        bits = plsc.load_gather(ed_v[s], [row2, jnp.full((16,), e, _i32)])
        a_vec = plsc.bitcast(bits, _f32)
        for g in range(D_C // 32):
          w = pkrows[b][e, pl.ds(g * 16, 16)]
          lo, hi = plsc.unpack(plsc.bitcast(w, jnp.bfloat16),
                               format=plsc.PackFormat.INTERLEAVED)
          rows[b].at[e, pl.ds(g * 32, 16)][...] = lo * a_vec
          rows[b].at[e, pl.ds(g * 32 + 16, 16)][...] = hi * a_vec

    plsc.subcore_barrier()

    for s in range(NIDX):
      istart(s, s)
    for b in range(NBUF):
      iwait(b, b)
      gstart(b, b)

    @pl.loop(0, NCHUNK, step=NIDX)
    def _chunks(j):
      for b in range(NIDX):
        c = j + b
        bb = b % NBUF
        gwait(bb, b)

        @pl.when(c - 2 >= 0)
        def _drain():
          swait(bb, (b - 2) % NIDX)

        scale(bb, b)
        sstart(bb, b)

        @pl.when(c + NBUF < NCHUNK)
        def _refill():
          iwait((b + NBUF) % NIDX, c + NBUF)
          gstart(bb, (b + NBUF) % NIDX)

        @pl.when(jnp.logical_and(c - 2 >= 0, c + 6 < NCHUNK))
        def _meta():
          istart((b + 6) % NIDX, c + 6)

    for x in (NCHUNK - 2, NCHUNK - 1):
      swait(x % NBUF, x % NIDX)
    plsc.subcore_barrier()
    pltpu.sync_copy(acc_sh.at[pl.ds(sid * RPS, RPS)],
                    out_h.at[cid, pl.ds(sid * RPS, RPS)])

  return k(ego_pk, ed)


def _dense_tc(ego, parts, Wg, bg, Wb, bb):
  blk = 1000
  grid = N_C // blk

  def body(ego_r, p_r, wg_r, bg_r, wb_r, bb_r, new_r, nrm_r):
    side = p_r[0] + p_r[1]
    e = ego_r[...]
    s = jnp.dot(e + side, wg_r[...], preferred_element_type=_f32,
                precision=lax.Precision.HIGHEST) + bg_r[...]
    s = jnp.where(s >= 0, s, 0.2 * s)
    m = jnp.dot(e * side, wb_r[...], preferred_element_type=_f32,
                precision=lax.Precision.HIGHEST) + bb_r[...]
    m = jnp.where(m >= 0, m, 0.2 * m)
    out = s + m
    new_r[...] = out
    norm = jnp.sqrt(jnp.sum(out * out, axis=1, keepdims=True))
    nrm_r[...] = out / jnp.maximum(norm, 1e-12)

  return pl.pallas_call(
      body,
      grid=(grid,),
      in_specs=[
          pl.BlockSpec((blk, D_C), lambda i: (i, 0)),
          pl.BlockSpec((NC, blk, D_C), lambda i: (0, i, 0)),
          pl.BlockSpec((D_C, D_C), lambda i: (0, 0)),
          pl.BlockSpec((1, D_C), lambda i: (0, 0)),
          pl.BlockSpec((D_C, D_C), lambda i: (0, 0)),
          pl.BlockSpec((1, D_C), lambda i: (0, 0)),
      ],
      out_specs=[
          pl.BlockSpec((blk, D_C), lambda i: (i, 0)),
          pl.BlockSpec((blk, D_C), lambda i: (i, 0)),
      ],
      out_shape=[
          jax.ShapeDtypeStruct((N_C, D_C), _f32),
          jax.ShapeDtypeStruct((N_C, D_C), _f32),
      ],
  )(ego, parts, Wg, bg, Wb, bb)


def _gather_sc(t0, t1, t2, u_t, p_t):
  @functools.partial(
      pl.kernel,
      out_type=(jax.ShapeDtypeStruct((3, B_C, D_C), _f32),
                jax.ShapeDtypeStruct((3, B_C, D_C), _f32)),
      mesh=_mesh,
      scratch_types=[
          pltpu.VMEM((BPT,), _i32),
          pltpu.VMEM((BPT, D_C), _f32),
          pltpu.SemaphoreType.DMA,
      ],
  )
  def k(t0_h, t1_h, t2_h, u_h, p_h, ue_h, pe_h, idx_v, rows_v, sem):
    cid = lax.axis_index("c")
    sid = lax.axis_index("s")
    wid = sid * NC + cid
    for idx_h, out_h in ((u_h, ue_h), (p_h, pe_h)):
      pltpu.sync_copy(idx_h.at[wid], idx_v)
      for t, t_h in enumerate((t0_h, t1_h, t2_h)):
        pltpu.async_copy(t_h.at[idx_v], rows_v, sem).wait()
        pltpu.sync_copy(rows_v, out_h.at[t, pl.ds(wid * BPT, BPT)])

  return k(t0, t1, t2, u_t, p_t)


def _score_tc(u_parts, p_parts):
  def body(u_r, p_r, o_r):
    acc = jnp.zeros((B_C, B_C), _f32)
    for t in range(3):
      acc = acc + lax.dot_general(
          u_r[t], p_r[t], (((1,), (1,)), ((), ())),
          preferred_element_type=_f32, precision=lax.Precision.HIGHEST)
    o_r[...] = acc

  return pl.pallas_call(
      body,
      out_shape=jax.ShapeDtypeStruct((B_C, B_C), _f32),
  )(u_parts, p_parts)


def _pack_table(x):
  bf = x.astype(jnp.bfloat16).reshape(N_C, D_C // 32, 2, 16)
  bf = bf.transpose(0, 1, 3, 2).reshape(N_C, DW, 2)
  return jax.lax.bitcast_convert_type(bf, _i32)


def kernel(users, pos_items, edge_index, A_values, user_embed, entity_embed,
           W_gc_0, b_gc_0, W_bi_0, b_bi_0, W_gc_1, b_gc_1, W_bi_1, b_bi_1):
  ego0 = jnp.concatenate([user_embed, entity_embed], axis=0)

  pad = E_PAD - E_C
  src_t = jnp.pad(edge_index[0].astype(_i32), (0, pad)).reshape(NW, NCHUNK, CH)
  dst_t = jnp.pad(edge_index[1].astype(_i32), (0, pad)).reshape(NW, NCHUNK, CH)
  a_bits = jax.lax.bitcast_convert_type(
      jnp.pad(A_values.astype(_f32), (0, pad)), _i32).reshape(NW, NCHUNK, CH)
  ed = jnp.stack([src_t, dst_t, a_bits], axis=2)

  parts1 = _side_sc(_pack_table(ego0), ed)
  ego1, n1 = _dense_tc(ego0, parts1, W_gc_0, b_gc_0, W_bi_0, b_bi_0)
  parts2 = _side_sc(_pack_table(ego1), ed)
  _, n2 = _dense_tc(ego1, parts2, W_gc_1, b_gc_1, W_bi_1, b_bi_1)

  u_t = users.astype(_i32).reshape(NW, BPT)
  p_t = (pos_items.astype(_i32) + N_USERS_C).reshape(NW, BPT)
  u_parts, p_parts = _gather_sc(ego0, n1, n2, u_t, p_t)
  return _score_tc(u_parts, p_parts)
